# baseline (speedup 1.0000x reference)
import functools

import jax
import jax.numpy as jnp
from jax.experimental import pallas as pl
from jax.experimental.pallas import tpu as pltpu

TOPK = 8
E = 64
D = 4096
N = 16384
BT = 1024


def _fused_kernel(x1_ref, x2_ref, w_ref, probs_ref, idx_ref, aux_ref,
                  cnt_acc, psum_acc):
    step = pl.program_id(0)
    nsteps = pl.num_programs(0)

    @pl.when(step == 0)
    def _init():
        cnt_acc[...] = jnp.zeros_like(cnt_acc)
        psum_acc[...] = jnp.zeros_like(psum_acc)

    w = w_ref[...]
    logits = (
        jnp.dot(x1_ref[...], w[:D // 2], preferred_element_type=jnp.float32)
        + jnp.dot(x2_ref[...], w[D // 2:], preferred_element_type=jnp.float32)
    )

    m = jnp.max(logits, axis=-1, keepdims=True)
    ex = jnp.exp(logits - m)
    z = jnp.sum(ex, axis=-1, keepdims=True)

    iota = jax.lax.broadcasted_iota(jnp.int32, ex.shape, 1)
    work = ex
    vals = []
    idxs = []
    disp = jnp.zeros_like(ex)
    for _ in range(TOPK):
        ik = jnp.argmax(work, axis=-1)[:, None]
        mk = jnp.max(work, axis=-1, keepdims=True)
        sel = iota == ik
        vals.append(mk)
        idxs.append(ik)
        disp = disp + sel.astype(jnp.float32)
        work = jnp.where(sel, 0.0, work)

    tope = jnp.concatenate(vals, axis=-1)
    probs_ref[...] = tope / jnp.sum(tope, axis=-1, keepdims=True)
    idx_ref[...] = jnp.concatenate(idxs, axis=-1)

    cnt_acc[...] += jnp.sum(disp, axis=0, keepdims=True)
    psum_acc[...] += jnp.sum(ex / z, axis=0, keepdims=True)

    @pl.when(step == nsteps - 1)
    def _fin():
        aux = jnp.sum(cnt_acc[...] * psum_acc[...]) * (
            float(E) / (float(N) * float(N)))
        aux_ref[...] = aux.reshape(1, 1)


@functools.partial(jax.jit)
def _run(input_matrix, W_router):
    grid = N // BT
    probs, idx, aux = pl.pallas_call(
        _fused_kernel,
        grid=(grid,),
        in_specs=[
            pl.BlockSpec((BT, D // 2), lambda i: (i, 0)),
            pl.BlockSpec((BT, D // 2), lambda i: (i, 1)),
            pl.BlockSpec((D, E), lambda i: (0, 0)),
        ],
        out_specs=[
            pl.BlockSpec((BT, TOPK), lambda i: (i, 0)),
            pl.BlockSpec((BT, TOPK), lambda i: (i, 0)),
            pl.BlockSpec((1, 1), lambda i: (0, 0)),
        ],
        out_shape=[
            jax.ShapeDtypeStruct((N, TOPK), jnp.float32),
            jax.ShapeDtypeStruct((N, TOPK), jnp.int32),
            jax.ShapeDtypeStruct((1, 1), jnp.float32),
        ],
        scratch_shapes=[
            pltpu.VMEM((1, E), jnp.float32),
            pltpu.VMEM((1, E), jnp.float32),
        ],
        compiler_params=pltpu.CompilerParams(
            dimension_semantics=("arbitrary",),
        ),
    )(input_matrix, input_matrix, W_router)
    return probs, idx, aux[0, 0]


def kernel(input_matrix, W_router):
    return _run(input_matrix, W_router)


# speedup vs baseline: 1.0086x; 1.0086x over previous
import functools

import jax
import jax.numpy as jnp
from jax.experimental import pallas as pl
from jax.experimental.pallas import tpu as pltpu

TOPK = 8
E = 64
D = 4096
N = 16384
BT = 1024


def _fused_kernel(x_ref, w_ref, probs_ref, idx_ref, aux_ref,
                  cnt_acc, psum_acc):
    step = pl.program_id(0)
    nsteps = pl.num_programs(0)

    @pl.when(step == 0)
    def _init():
        cnt_acc[...] = jnp.zeros_like(cnt_acc)
        psum_acc[...] = jnp.zeros_like(psum_acc)

    x = x_ref[...]
    w = w_ref[...]
    logits = jnp.dot(x, w, preferred_element_type=jnp.float32)

    m = jnp.max(logits, axis=-1, keepdims=True)
    ex = jnp.exp(logits - m)
    z = jnp.sum(ex, axis=-1, keepdims=True)

    iota = jax.lax.broadcasted_iota(jnp.int32, ex.shape, 1)
    work = ex
    vals = []
    idxs = []
    for k in range(TOPK):
        ik = jnp.argmax(work, axis=-1)[:, None]
        if k == 0:
            mk = jnp.ones_like(z)
        else:
            mk = jnp.max(work, axis=-1, keepdims=True)
        vals.append(mk)
        idxs.append(ik)
        work = jnp.where(iota == ik, 0.0, work)

    tope = jnp.concatenate(vals, axis=-1)
    probs_ref[...] = tope * (1.0 / jnp.sum(tope, axis=-1, keepdims=True))
    idx_ref[...] = jnp.concatenate(idxs, axis=-1)

    disp = (work != ex).astype(jnp.float32)
    cnt_acc[...] += jnp.sum(disp, axis=0, keepdims=True)
    psum_acc[...] += jnp.sum(ex * (1.0 / z), axis=0, keepdims=True)

    @pl.when(step == nsteps - 1)
    def _fin():
        aux = jnp.sum(cnt_acc[...] * psum_acc[...]) * (
            float(E) / (float(N) * float(N)))
        aux_ref[...] = aux.reshape(1, 1)


@functools.partial(jax.jit)
def _run(input_matrix, W_router):
    grid = N // BT
    probs, idx, aux = pl.pallas_call(
        _fused_kernel,
        grid=(grid,),
        in_specs=[
            pl.BlockSpec((BT, D), lambda i: (i, 0)),
            pl.BlockSpec((D, E), lambda i: (0, 0)),
        ],
        out_specs=[
            pl.BlockSpec((BT, TOPK), lambda i: (i, 0)),
            pl.BlockSpec((BT, TOPK), lambda i: (i, 0)),
            pl.BlockSpec((1, 1), lambda i: (0, 0)),
        ],
        out_shape=[
            jax.ShapeDtypeStruct((N, TOPK), jnp.float32),
            jax.ShapeDtypeStruct((N, TOPK), jnp.int32),
            jax.ShapeDtypeStruct((1, 1), jnp.float32),
        ],
        scratch_shapes=[
            pltpu.VMEM((1, E), jnp.float32),
            pltpu.VMEM((1, E), jnp.float32),
        ],
        compiler_params=pltpu.CompilerParams(
            dimension_semantics=("arbitrary",),
        ),
    )(input_matrix, W_router)
    return probs, idx, aux[0, 0]


def kernel(input_matrix, W_router):
    return _run(input_matrix, W_router)


# speedup vs baseline: 1.2663x; 1.2556x over previous
import functools

import jax
import jax.numpy as jnp
from jax.experimental import pallas as pl
from jax.experimental.pallas import tpu as pltpu

TOPK = 8
E = 64
D = 4096
N = 16384
BT = 1024


def _fused_kernel(x_ref, w_ref, probs_ref, idx_ref, aux_ref,
                  cnt_acc, psum_acc):
    step = pl.program_id(0)
    nsteps = pl.num_programs(0)

    @pl.when(step == 0)
    def _init():
        cnt_acc[...] = jnp.zeros_like(cnt_acc)
        psum_acc[...] = jnp.zeros_like(psum_acc)

    x = x_ref[...]
    w = w_ref[...]
    logits = jnp.dot(x, w, preferred_element_type=jnp.float32)

    lt = jnp.transpose(logits)
    m = jnp.max(lt, axis=0, keepdims=True)
    ex = jnp.exp(lt - m)
    z = jnp.sum(ex, axis=0, keepdims=True)

    iota = jax.lax.broadcasted_iota(jnp.int32, ex.shape, 0)
    work = ex
    vals = []
    idxs = []
    for k in range(TOPK):
        if k == 0:
            mk = jnp.ones_like(z)
        else:
            mk = jnp.max(work, axis=0, keepdims=True)
        hit = work == mk
        ik = jnp.min(jnp.where(hit, iota, E), axis=0, keepdims=True)
        vals.append(mk)
        idxs.append(ik)
        work = jnp.where(iota == ik, 0.0, work)

    tope = jnp.concatenate(vals, axis=0)
    probs_ref[...] = tope * (1.0 / jnp.sum(tope, axis=0, keepdims=True))
    idx_ref[...] = jnp.concatenate(idxs, axis=0)

    disp = (work != ex).astype(jnp.float32)
    cnt_acc[...] += jnp.sum(disp, axis=1, keepdims=True)
    psum_acc[...] += jnp.sum(ex * (1.0 / z), axis=1, keepdims=True)

    @pl.when(step == nsteps - 1)
    def _fin():
        aux = jnp.sum(cnt_acc[...] * psum_acc[...]) * (
            float(E) / (float(N) * float(N)))
        aux_ref[...] = aux.reshape(1, 1)


@functools.partial(jax.jit)
def _run(input_matrix, W_router):
    grid = N // BT
    probs, idx, aux = pl.pallas_call(
        _fused_kernel,
        grid=(grid,),
        in_specs=[
            pl.BlockSpec((BT, D), lambda i: (i, 0)),
            pl.BlockSpec((D, E), lambda i: (0, 0)),
        ],
        out_specs=[
            pl.BlockSpec((TOPK, BT), lambda i: (0, i)),
            pl.BlockSpec((TOPK, BT), lambda i: (0, i)),
            pl.BlockSpec((1, 1), lambda i: (0, 0)),
        ],
        out_shape=[
            jax.ShapeDtypeStruct((TOPK, N), jnp.float32),
            jax.ShapeDtypeStruct((TOPK, N), jnp.int32),
            jax.ShapeDtypeStruct((1, 1), jnp.float32),
        ],
        scratch_shapes=[
            pltpu.VMEM((E, 1), jnp.float32),
            pltpu.VMEM((E, 1), jnp.float32),
        ],
        compiler_params=pltpu.CompilerParams(
            dimension_semantics=("arbitrary",),
        ),
    )(input_matrix, W_router)
    return probs.T, idx.T, aux[0, 0]


def kernel(input_matrix, W_router):
    return _run(input_matrix, W_router)
